# baseline (speedup 1.0000x reference)
import functools

import numpy as np
import jax
import jax.numpy as jnp
from jax import lax
from jax.experimental import pallas as pl
from jax.experimental.pallas import tpu as pltpu
from jax.experimental.pallas import tpu_sc as plsc

_NC = 2
_NS = 16
_NW = _NC * _NS

_CHUNK = 80
_NSLOT = 3
_STRIPES = 4
_BB = 8


def _positional_encoding_np(seq_len: int, d_model: int) -> np.ndarray:
    position = np.arange(seq_len, dtype=np.float32)[:, None]
    div_term = np.exp(
        np.arange(0, d_model, 2, dtype=np.float32) * (-(np.log(10000.0) / d_model))
    )
    pe = np.zeros((seq_len, d_model), dtype=np.float32)
    pe[:, 0::2] = np.sin(position * div_term)
    pe[:, 1::2] = np.cos(position * div_term)
    return pe


def _sc_gather(tok_idx, token_table):
    n_chunks, per_w = tok_idx.shape[1], tok_idx.shape[1] * _CHUNK
    d = token_table.shape[1]
    n = _NW * per_w
    n_ring = (n_chunks // _NSLOT) * _NSLOT

    mesh = plsc.VectorSubcoreMesh(core_axis_name="c", subcore_axis_name="s")

    @functools.partial(
        pl.kernel,
        mesh=mesh,
        out_type=jax.ShapeDtypeStruct((n, d), jnp.float32),
        scratch_types=[
            pltpu.VMEM((n_chunks, _CHUNK), jnp.int32),
        ]
        + [pltpu.VMEM((_CHUNK, d), jnp.float32)] * _NSLOT
        + [pltpu.SemaphoreType.DMA] * (2 * _NSLOT),
    )
    def k(tok_idx_hbm, table_hbm, out_hbm, tidx_v, *bufs_and_sems):
        toks = bufs_and_sems[0:_NSLOT]
        gsems = bufs_and_sems[_NSLOT:2 * _NSLOT]
        osems = bufs_and_sems[2 * _NSLOT:]
        wid = lax.axis_index("s") * _NC + lax.axis_index("c")
        base = wid * per_w
        pltpu.sync_copy(tok_idx_hbm.at[wid], tidx_v)

        def issue_g(c, s):
            pltpu.async_copy(table_hbm.at[tidx_v.at[c]], toks[s], gsems[s])

        def wait_g(s):
            pltpu.make_async_copy(
                table_hbm.at[tidx_v.at[0]], toks[s], gsems[s]).wait()

        def start_o(c, s):
            pltpu.async_copy(
                toks[s], out_hbm.at[pl.ds(base + c * _CHUNK, _CHUNK)], osems[s])

        def wait_o(s):
            pltpu.make_async_copy(
                toks[s], out_hbm.at[pl.ds(0, _CHUNK)], osems[s]).wait()

        issue_g(0, 0)
        issue_g(1, 1)

        def process(c, s):
            wait_g(s)
            start_o(c, s)

            @pl.when((c >= 1) & (c + 2 < n_chunks))
            def _():
                wait_o((s + 2) % _NSLOT)

            @pl.when(c + 2 < n_chunks)
            def _():
                issue_g(c + 2, (s + 2) % _NSLOT)

        def ring_body(q, carry):
            for b in range(_NSLOT):
                process(_NSLOT * q + b, b)
            return carry

        lax.fori_loop(0, n_chunks // _NSLOT, ring_body, 0)
        for c in range(n_ring, n_chunks):
            s = c % _NSLOT
            wait_g(s)
            start_o(c, s)
        for s in range(_NSLOT):
            wait_o(s)

    return k(tok_idx, token_table)


def _tc_add_kernel(g_ref, seg_ref, pe_ref, st_ref, out_ref):
    g = g_ref[...]
    seg = seg_ref[...][:, :, None]
    s1 = st_ref[1][None, None, :]
    s2 = st_ref[2][None, None, :]
    add = pe_ref[...][None, :, :]
    add = add + jnp.where(seg == 1, s1, 0.0) + jnp.where(seg == 2, s2, 0.0)
    out_ref[...] = g + add


def _tc_add(g, segment_label, pe, segment_table):
    b, l, d = g.shape
    grid = (b // _BB,)
    return pl.pallas_call(
        _tc_add_kernel,
        grid=grid,
        in_specs=[
            pl.BlockSpec((_BB, l, d), lambda i: (i, 0, 0)),
            pl.BlockSpec((_BB, l), lambda i: (i, 0)),
            pl.BlockSpec((l, d), lambda i: (0, 0)),
            pl.BlockSpec((3, d), lambda i: (0, 0)),
        ],
        out_specs=pl.BlockSpec((_BB, l, d), lambda i: (i, 0, 0)),
        out_shape=jax.ShapeDtypeStruct((b, l, d), jnp.float32),
    )(g, segment_label, pe, segment_table)


def kernel(sequence, segment_label, token_table, segment_table):
    b, l = sequence.shape
    d = token_table.shape[1]

    pe = jnp.asarray(_positional_encoding_np(l, d))
    seg32 = segment_label.astype(jnp.int32)
    tok32 = sequence.astype(jnp.int32)

    bs = b // _STRIPES
    n_s = bs * l
    rows_per_w = n_s // _NW

    outs = []
    for t in range(_STRIPES):
        tok_idx = tok32[t * bs:(t + 1) * bs].reshape(n_s)
        tok_idx = tok_idx.reshape(_NW, rows_per_w // _CHUNK, _CHUNK)
        g = _sc_gather(tok_idx, token_table).reshape(bs, l, d)
        outs.append(_tc_add(g, seg32[t * bs:(t + 1) * bs], pe, segment_table))
    return jnp.concatenate(outs, axis=0)


# speedup vs baseline: 1.3941x; 1.3941x over previous
import functools

import numpy as np
import jax
import jax.numpy as jnp
from jax import lax
from jax.experimental import pallas as pl
from jax.experimental.pallas import tpu as pltpu
from jax.experimental.pallas import tpu_sc as plsc

_NC = 2
_NS = 16
_NW = _NC * _NS
_LANES = 16

_CHUNK = 128
_NSLOT = 3


def _positional_encoding_np(seq_len: int, d_model: int) -> np.ndarray:
    position = np.arange(seq_len, dtype=np.float32)[:, None]
    div_term = np.exp(
        np.arange(0, d_model, 2, dtype=np.float32) * (-(np.log(10000.0) / d_model))
    )
    pe = np.zeros((seq_len, d_model), dtype=np.float32)
    pe[:, 0::2] = np.sin(position * div_term)
    pe[:, 1::2] = np.cos(position * div_term)
    return pe


def _sc_embed(tok_idx, comb_idx, token_table, comb_table):
    n_chunks, per_w = tok_idx.shape[1], tok_idx.shape[1] * _CHUNK
    d = token_table.shape[1]
    n = _NW * per_w
    n_ring = (n_chunks // _NSLOT) * _NSLOT

    mesh = plsc.VectorSubcoreMesh(core_axis_name="c", subcore_axis_name="s")

    @functools.partial(
        pl.kernel,
        mesh=mesh,
        out_type=jax.ShapeDtypeStruct((n, d), jnp.float32),
        scratch_types=[
            pltpu.VMEM((n_chunks, _CHUNK), jnp.int32),
            pltpu.VMEM((n_chunks, _CHUNK), jnp.int32),
        ]
        + [pltpu.VMEM((_CHUNK, d), jnp.float32)] * (2 * _NSLOT)
        + [pltpu.SemaphoreType.DMA] * (2 * _NSLOT),
    )
    def k(tok_idx_hbm, comb_idx_hbm, table_hbm, comb_hbm, out_hbm,
          tidx_v, cidx_v, *bufs_and_sems):
        toks = bufs_and_sems[0:_NSLOT]
        combs = bufs_and_sems[_NSLOT:2 * _NSLOT]
        gsems = bufs_and_sems[2 * _NSLOT:3 * _NSLOT]
        osems = bufs_and_sems[3 * _NSLOT:]
        wid = lax.axis_index("s") * _NC + lax.axis_index("c")
        base = wid * per_w
        pltpu.sync_copy(tok_idx_hbm.at[wid], tidx_v)
        pltpu.sync_copy(comb_idx_hbm.at[wid], cidx_v)

        def issue_g(c, s):
            pltpu.async_copy(table_hbm.at[tidx_v.at[c]], toks[s], gsems[s])
            pltpu.async_copy(comb_hbm.at[cidx_v.at[c]], combs[s], gsems[s])

        def wait_g(s):
            pltpu.make_async_copy(
                table_hbm.at[tidx_v.at[0]], toks[s], gsems[s]).wait()
            pltpu.make_async_copy(
                comb_hbm.at[cidx_v.at[0]], combs[s], gsems[s]).wait()

        def start_o(c, s):
            pltpu.async_copy(
                toks[s], out_hbm.at[pl.ds(base + c * _CHUNK, _CHUNK)], osems[s])

        def wait_o(s):
            pltpu.make_async_copy(
                toks[s], out_hbm.at[pl.ds(0, _CHUNK)], osems[s]).wait()

        def add_chunk(tok_b, comb_b):
            def add_body(j, carry):
                for r in range(2):
                    i = 2 * j + r
                    for kk in range(d // _LANES):
                        sl = pl.ds(kk * _LANES, _LANES)
                        plsc.addupdate(tok_b.at[i, sl], comb_b[i, sl])
                return carry
            lax.fori_loop(0, _CHUNK // 2, add_body, 0)

        def process(c, s):
            wait_g(s)
            add_chunk(toks[s], combs[s])
            start_o(c, s)

            @pl.when((c >= 1) & (c + 2 < n_chunks))
            def _():
                wait_o((s + 2) % _NSLOT)

            @pl.when(c + 2 < n_chunks)
            def _():
                issue_g(c + 2, (s + 2) % _NSLOT)

        issue_g(0, 0)
        issue_g(1, 1)

        def ring_body(q, carry):
            for b in range(_NSLOT):
                process(_NSLOT * q + b, b)
            return carry

        lax.fori_loop(0, n_chunks // _NSLOT, ring_body, 0)
        for c in range(n_ring, n_chunks):
            s = c % _NSLOT
            wait_g(s)
            add_chunk(toks[s], combs[s])
            start_o(c, s)
        for s in range(_NSLOT):
            wait_o(s)

    return k(tok_idx, comb_idx, token_table, comb_table)


def kernel(sequence, segment_label, token_table, segment_table):
    b, l = sequence.shape
    d = token_table.shape[1]
    n = b * l

    pe = jnp.asarray(_positional_encoding_np(l, d))
    comb = (pe[:, None, :] + segment_table[None, :, :]).reshape(l * 3, d)

    pos3 = (jnp.arange(l, dtype=jnp.int32) * 3)[None, :]
    comb_idx = (pos3 + segment_label.astype(jnp.int32)).reshape(n)
    tok_idx = sequence.astype(jnp.int32).reshape(n)

    rows_per_w = n // _NW
    tok_idx = tok_idx.reshape(_NW, rows_per_w // _CHUNK, _CHUNK)
    comb_idx = comb_idx.reshape(_NW, rows_per_w // _CHUNK, _CHUNK)

    out = _sc_embed(tok_idx, comb_idx, token_table, comb)
    return out.reshape(b, l, d)
